# baseline (speedup 1.0000x reference)
import functools

import jax
import jax.numpy as jnp
from jax import lax
from jax.experimental import pallas as pl
from jax.experimental.pallas import tpu as pltpu
from jax.experimental.pallas import tpu_sc as plsc

N = 1024
E = 16384
NC = 2
NS = 16
L = 16
NW = NC * NS
EPW = E // NW
G = EPW // L
F = 4 * N

TILE_E = 256
K_STEPS = E // TILE_E


def _edge_partials_sc(t, src, dst, a_flat, b_flat, w2s_flat):
    mesh = plsc.VectorSubcoreMesh(core_axis_name="c", subcore_axis_name="s",
                                  num_cores=NC, num_subcores=NS)

    @functools.partial(
        pl.kernel,
        out_type=jax.ShapeDtypeStruct((NW * F,), jnp.float32),
        mesh=mesh,
        compiler_params=pltpu.CompilerParams(needs_layout_passes=False),
        scratch_types=[
            pltpu.VMEM((N,), jnp.int32),
            pltpu.VMEM((EPW,), jnp.int32),
            pltpu.VMEM((EPW,), jnp.int32),
            pltpu.VMEM((N * L,), jnp.float32),
            pltpu.VMEM((N * L,), jnp.float32),
            pltpu.VMEM((48 * L,), jnp.float32),
            pltpu.VMEM((L * F,), jnp.float32),
            pltpu.VMEM((F,), jnp.float32),
        ],
    )
    def sc_kernel(t_hbm, src_hbm, dst_hbm, a_hbm, b_hbm, w2_hbm, out_hbm,
                  t_v, src_v, dst_v, a_v, b_v, w2_v, acc_v, red_v):
        wid = lax.axis_index("s") * NC + lax.axis_index("c")
        base = wid * EPW
        pltpu.sync_copy(t_hbm, t_v)
        pltpu.sync_copy(src_hbm.at[pl.ds(base, EPW)], src_v)
        pltpu.sync_copy(dst_hbm.at[pl.ds(base, EPW)], dst_v)
        pltpu.sync_copy(a_hbm, a_v)
        pltpu.sync_copy(b_hbm, b_v)
        pltpu.sync_copy(w2_hbm, w2_v)

        zero = jnp.zeros((L,), jnp.float32)

        def zbody(j, carry):
            for r in range(L):
                acc_v[pl.ds(j * (L * L) + r * L, L)] = zero
            return carry

        lax.fori_loop(0, (L * F) // (L * L), zbody, 0)

        lanes = lax.iota(jnp.int32, L)
        lane_off = lanes * F
        lane_idx = lanes * G

        def bf16_round(x):
            bits = plsc.bitcast(x, jnp.int32)
            rounded = (bits + 0x7FFF + ((bits >> 16) & 1)) & ~0xFFFF
            return plsc.bitcast(rounded, jnp.float32)

        def tree_sum(ps):
            while len(ps) > 1:
                nxt = [ps[i] + ps[i + 1] for i in range(0, len(ps) - 1, 2)]
                if len(ps) % 2:
                    nxt.append(ps[-1])
                ps = nxt
            return ps[0]

        def gbody(g, carry):
            ev = lane_idx + g
            srcv = plsc.load_gather(src_v, [ev])
            dstv = plsc.load_gather(dst_v, [ev])
            tdv = plsc.load_gather(t_v, [dstv]) * L
            tsv = plsc.load_gather(t_v, [srcv]) * L
            h = []
            for j in range(10):
                aj = plsc.load_gather(a_v, [tdv + j])
                bj = plsc.load_gather(b_v, [tsv + j])
                h.append(bf16_round(jnp.maximum(aj + bj, 0.0)))
            logit = []
            for c in range(4):
                w2c = [w2_v[pl.ds((j * 4 + c) * L, L)] for j in range(10)]
                prods = [h[j] * w2c[j] for j in range(10)]
                logit.append(tree_sum(prods) + w2_v[pl.ds((40 + c) * L, L)])
            m = jnp.maximum(jnp.maximum(logit[0], logit[1]),
                            jnp.maximum(logit[2], logit[3]))
            ex = [jnp.exp(x - m) for x in logit]
            inv = 1.0 / ((ex[0] + ex[1]) + (ex[2] + ex[3]))
            d4 = lane_off + dstv * 4
            for c in range(4):
                plsc.addupdate_scatter(acc_v, [d4 + c], ex[c] * inv)
            return carry

        lax.fori_loop(0, G, gbody, 0)

        def rbody(j, carry):
            v = acc_v[pl.ds(j * L, L)]
            for r in range(1, L):
                v = v + acc_v[pl.ds(r * F + j * L, L)]
            red_v[pl.ds(j * L, L)] = v
            return carry

        lax.fori_loop(0, F // L, rbody, 0)
        pltpu.sync_copy(red_v, out_hbm.at[pl.ds(wid * F, F)])

    return sc_kernel(t, src, dst, a_flat, b_flat, w2s_flat)


def _heads_tc(partials, wp, bp2, wv1, bv1, wv2, bv2, wv3, bv3, wv4, bv4):
    def body(part_ref, wp_ref, bp_ref, wv1_ref, bv1_ref, wv2_ref, bv2_ref,
             wv3_ref, bv3_ref, wv4_ref, bv4_ref, pol_ref, val_ref, flat_s):
        k = pl.program_id(0)

        bf = jnp.bfloat16

        @pl.when(k == 0)
        def _():
            parts = part_ref[...]
            flat = parts[0:1, :]
            for w in range(1, NW):
                flat = flat + parts[w:w + 1, :]
            flat_s[...] = flat.astype(bf)
            v = jnp.maximum(jnp.dot(flat.astype(bf), wv1_ref[...].astype(bf),
                                    preferred_element_type=jnp.float32)
                            + bv1_ref[...], 0.0)
            v = jnp.maximum(jnp.dot(v.astype(bf), wv2_ref[...].astype(bf),
                                    preferred_element_type=jnp.float32)
                            + bv2_ref[...], 0.0)
            v = jnp.maximum(jnp.dot(v.astype(bf), wv3_ref[...].astype(bf),
                                    preferred_element_type=jnp.float32)
                            + bv3_ref[...], 0.0)
            prods = v * wv4_ref[...]
            ps = [prods[:, i:i + 1] for i in range(16)]
            while len(ps) > 1:
                ps = [ps[i] + ps[i + 1] for i in range(0, len(ps) - 1, 2)] \
                    + ([ps[-1]] if len(ps) % 2 else [])
            val_ref[...] = ps[0] + bv4_ref[...]

        logits = jnp.dot(flat_s[...], wp_ref[...].astype(bf),
                         preferred_element_type=jnp.float32) + bp_ref[...]
        pol_ref[:, pl.ds(k * TILE_E, TILE_E)] = logits

        @pl.when(k == K_STEPS - 1)
        def _():
            full = pol_ref[...]
            m = jnp.max(full)
            ex = jnp.exp(full - m)
            pol_ref[...] = ex / jnp.sum(ex)

    policy, value = pl.pallas_call(
        body,
        grid=(K_STEPS,),
        in_specs=[
            pl.BlockSpec((NW, F), lambda k: (0, 0)),
            pl.BlockSpec((F, TILE_E), lambda k: (0, k)),
            pl.BlockSpec((1, TILE_E), lambda k: (0, k)),
            pl.BlockSpec((F, 64), lambda k: (0, 0)),
            pl.BlockSpec((1, 64), lambda k: (0, 0)),
            pl.BlockSpec((64, 32), lambda k: (0, 0)),
            pl.BlockSpec((1, 32), lambda k: (0, 0)),
            pl.BlockSpec((32, 16), lambda k: (0, 0)),
            pl.BlockSpec((1, 16), lambda k: (0, 0)),
            pl.BlockSpec((1, 16), lambda k: (0, 0)),
            pl.BlockSpec((1, 1), lambda k: (0, 0)),
        ],
        out_specs=[
            pl.BlockSpec((1, E), lambda k: (0, 0)),
            pl.BlockSpec((1, 1), lambda k: (0, 0)),
        ],
        out_shape=[
            jax.ShapeDtypeStruct((1, E), jnp.float32),
            jax.ShapeDtypeStruct((1, 1), jnp.float32),
        ],
        scratch_shapes=[pltpu.VMEM((1, F), jnp.bfloat16)],
    )(partials, wp, bp2, wv1, bv1, wv2, bv2, wv3, bv3, wv4, bv4)
    return policy, value


def kernel(target_nodes, edge_index, W1, b1, W2, b2, Wp, bp,
           Wv1, bv1, Wv2, bv2, Wv3, bv3, Wv4, bv4):
    t = target_nodes.astype(jnp.int32)
    src = edge_index[0].astype(jnp.int32)
    dst = edge_index[1].astype(jnp.int32)

    bf = jnp.bfloat16
    w1a = W1[:N].astype(bf).astype(jnp.float32)
    w1b = W1[N:].astype(bf).astype(jnp.float32)
    a_flat = jnp.pad((w1a - w1b) + b1[None, :],
                     ((0, 0), (0, L - 10))).reshape(N * L)
    b_flat = jnp.pad(w1b, ((0, 0), (0, L - 10))).reshape(N * L)
    w2r = W2.astype(bf).astype(jnp.float32)
    w2s_flat = (jnp.concatenate(
        [w2r.reshape(40, 1), b2.reshape(4, 1), jnp.zeros((4, 1), jnp.float32)]
    ) * jnp.ones((1, L), jnp.float32)).reshape(48 * L)

    partials = _edge_partials_sc(t, src, dst, a_flat, b_flat, w2s_flat)

    policy, value = _heads_tc(
        partials.reshape(NW, F), Wp, bp.reshape(1, E),
        Wv1, bv1.reshape(1, 64), Wv2, bv2.reshape(1, 32),
        Wv3, bv3.reshape(1, 16), Wv4.reshape(1, 16), bv4.reshape(1, 1))
    return policy.reshape(E), value.reshape(1)


# speedup vs baseline: 1.0549x; 1.0549x over previous
import functools

import jax
import jax.numpy as jnp
from jax import lax
from jax.experimental import pallas as pl
from jax.experimental.pallas import tpu as pltpu
from jax.experimental.pallas import tpu_sc as plsc

N = 1024
E = 16384
NC = 2
NS = 16
L = 16
NW = NC * NS
EPW = E // NW
G = EPW // L
F = 4 * N

TILE_E = 512
K_STEPS = E // TILE_E


def _edge_partials_sc(t, src, dst, a_flat, b_flat, w2s_flat):
    mesh = plsc.VectorSubcoreMesh(core_axis_name="c", subcore_axis_name="s",
                                  num_cores=NC, num_subcores=NS)

    @functools.partial(
        pl.kernel,
        out_type=jax.ShapeDtypeStruct((NW * F,), jnp.float32),
        mesh=mesh,
        compiler_params=pltpu.CompilerParams(needs_layout_passes=False),
        scratch_types=[
            pltpu.VMEM((N,), jnp.int32),
            pltpu.VMEM((EPW,), jnp.int32),
            pltpu.VMEM((EPW,), jnp.int32),
            pltpu.VMEM((N * L,), jnp.float32),
            pltpu.VMEM((N * L,), jnp.float32),
            pltpu.VMEM((48 * L,), jnp.float32),
            pltpu.VMEM((L * F,), jnp.float32),
            pltpu.VMEM((F,), jnp.float32),
        ],
    )
    def sc_kernel(t_hbm, src_hbm, dst_hbm, a_hbm, b_hbm, w2_hbm, out_hbm,
                  t_v, src_v, dst_v, a_v, b_v, w2_v, acc_v, red_v):
        wid = lax.axis_index("s") * NC + lax.axis_index("c")
        base = wid * EPW
        pltpu.sync_copy(t_hbm, t_v)
        pltpu.sync_copy(src_hbm.at[pl.ds(base, EPW)], src_v)
        pltpu.sync_copy(dst_hbm.at[pl.ds(base, EPW)], dst_v)
        pltpu.sync_copy(a_hbm, a_v)
        pltpu.sync_copy(b_hbm, b_v)
        pltpu.sync_copy(w2_hbm, w2_v)

        zero = jnp.zeros((L,), jnp.float32)

        def zbody(j, carry):
            for r in range(L):
                acc_v[pl.ds(j * (L * L) + r * L, L)] = zero
            return carry

        lax.fori_loop(0, (L * F) // (L * L), zbody, 0)

        lanes = lax.iota(jnp.int32, L)
        lane_off = lanes * F
        lane_idx = lanes * G

        def bf16_round(x):
            bits = plsc.bitcast(x, jnp.int32)
            rounded = (bits + 0x7FFF + ((bits >> 16) & 1)) & ~0xFFFF
            return plsc.bitcast(rounded, jnp.float32)

        def tree_sum(ps):
            while len(ps) > 1:
                nxt = [ps[i] + ps[i + 1] for i in range(0, len(ps) - 1, 2)]
                if len(ps) % 2:
                    nxt.append(ps[-1])
                ps = nxt
            return ps[0]

        def gstep(g):
            ev = lane_idx + g
            srcv = plsc.load_gather(src_v, [ev])
            dstv = plsc.load_gather(dst_v, [ev])
            tdv = plsc.load_gather(t_v, [dstv]) * L
            tsv = plsc.load_gather(t_v, [srcv]) * L
            h = []
            for j in range(10):
                aj = plsc.load_gather(a_v, [tdv + j])
                bj = plsc.load_gather(b_v, [tsv + j])
                h.append(bf16_round(jnp.maximum(aj + bj, 0.0)))
            logit = []
            for c in range(4):
                w2c = [w2_v[pl.ds((j * 4 + c) * L, L)] for j in range(10)]
                prods = [h[j] * w2c[j] for j in range(10)]
                logit.append(tree_sum(prods) + w2_v[pl.ds((40 + c) * L, L)])
            m = jnp.maximum(jnp.maximum(logit[0], logit[1]),
                            jnp.maximum(logit[2], logit[3]))
            ex = [jnp.exp(x - m) for x in logit]
            inv = 1.0 / ((ex[0] + ex[1]) + (ex[2] + ex[3]))
            d4 = lane_off + dstv * 4
            for c in range(4):
                plsc.addupdate_scatter(acc_v, [d4 + c], ex[c] * inv)

        def gbody(i, carry):
            gstep(i * 2)
            gstep(i * 2 + 1)
            return carry

        lax.fori_loop(0, G // 2, gbody, 0)

        def rbody(j, carry):
            v = acc_v[pl.ds(j * L, L)]
            for r in range(1, L):
                v = v + acc_v[pl.ds(r * F + j * L, L)]
            red_v[pl.ds(j * L, L)] = v
            return carry

        lax.fori_loop(0, F // L, rbody, 0)
        pltpu.sync_copy(red_v, out_hbm.at[pl.ds(wid * F, F)])

    return sc_kernel(t, src, dst, a_flat, b_flat, w2s_flat)


def _heads_tc(partials, wp, bp2, wv1, bv1, wv2, bv2, wv3, bv3, wv4, bv4):
    def body(part_ref, wp_ref, bp_ref, wv1_ref, bv1_ref, wv2_ref, bv2_ref,
             wv3_ref, bv3_ref, wv4_ref, bv4_ref, pol_ref, val_ref, flat_s):
        k = pl.program_id(0)

        bf = jnp.bfloat16

        @pl.when(k == 0)
        def _():
            parts = part_ref[...]
            flat = parts[0:1, :]
            for w in range(1, NW):
                flat = flat + parts[w:w + 1, :]
            flat_s[...] = flat.astype(bf)
            v = jnp.maximum(jnp.dot(flat.astype(bf), wv1_ref[...].astype(bf),
                                    preferred_element_type=jnp.float32)
                            + bv1_ref[...], 0.0)
            v = jnp.maximum(jnp.dot(v.astype(bf), wv2_ref[...].astype(bf),
                                    preferred_element_type=jnp.float32)
                            + bv2_ref[...], 0.0)
            v = jnp.maximum(jnp.dot(v.astype(bf), wv3_ref[...].astype(bf),
                                    preferred_element_type=jnp.float32)
                            + bv3_ref[...], 0.0)
            prods = v * wv4_ref[...]
            ps = [prods[:, i:i + 1] for i in range(16)]
            while len(ps) > 1:
                ps = [ps[i] + ps[i + 1] for i in range(0, len(ps) - 1, 2)] \
                    + ([ps[-1]] if len(ps) % 2 else [])
            val_ref[...] = ps[0] + bv4_ref[...]

        logits = jnp.dot(flat_s[...], wp_ref[...].astype(bf),
                         preferred_element_type=jnp.float32) + bp_ref[...]
        pol_ref[:, pl.ds(k * TILE_E, TILE_E)] = logits

        @pl.when(k == K_STEPS - 1)
        def _():
            full = pol_ref[...]
            m = jnp.max(full)
            ex = jnp.exp(full - m)
            pol_ref[...] = ex / jnp.sum(ex)

    policy, value = pl.pallas_call(
        body,
        grid=(K_STEPS,),
        in_specs=[
            pl.BlockSpec((NW, F), lambda k: (0, 0)),
            pl.BlockSpec((F, TILE_E), lambda k: (0, k)),
            pl.BlockSpec((1, TILE_E), lambda k: (0, k)),
            pl.BlockSpec((F, 64), lambda k: (0, 0)),
            pl.BlockSpec((1, 64), lambda k: (0, 0)),
            pl.BlockSpec((64, 32), lambda k: (0, 0)),
            pl.BlockSpec((1, 32), lambda k: (0, 0)),
            pl.BlockSpec((32, 16), lambda k: (0, 0)),
            pl.BlockSpec((1, 16), lambda k: (0, 0)),
            pl.BlockSpec((1, 16), lambda k: (0, 0)),
            pl.BlockSpec((1, 1), lambda k: (0, 0)),
        ],
        out_specs=[
            pl.BlockSpec((1, E), lambda k: (0, 0)),
            pl.BlockSpec((1, 1), lambda k: (0, 0)),
        ],
        out_shape=[
            jax.ShapeDtypeStruct((1, E), jnp.float32),
            jax.ShapeDtypeStruct((1, 1), jnp.float32),
        ],
        scratch_shapes=[pltpu.VMEM((1, F), jnp.bfloat16)],
    )(partials, wp, bp2, wv1, bv1, wv2, bv2, wv3, bv3, wv4, bv4)
    return policy, value


def kernel(target_nodes, edge_index, W1, b1, W2, b2, Wp, bp,
           Wv1, bv1, Wv2, bv2, Wv3, bv3, Wv4, bv4):
    t = target_nodes.astype(jnp.int32)
    src = edge_index[0].astype(jnp.int32)
    dst = edge_index[1].astype(jnp.int32)

    bf = jnp.bfloat16
    w1a = W1[:N].astype(bf).astype(jnp.float32)
    w1b = W1[N:].astype(bf).astype(jnp.float32)
    a_flat = jnp.pad((w1a - w1b) + b1[None, :],
                     ((0, 0), (0, L - 10))).reshape(N * L)
    b_flat = jnp.pad(w1b, ((0, 0), (0, L - 10))).reshape(N * L)
    w2r = W2.astype(bf).astype(jnp.float32)
    w2s_flat = (jnp.concatenate(
        [w2r.reshape(40, 1), b2.reshape(4, 1), jnp.zeros((4, 1), jnp.float32)]
    ) * jnp.ones((1, L), jnp.float32)).reshape(48 * L)

    partials = _edge_partials_sc(t, src, dst, a_flat, b_flat, w2s_flat)

    policy, value = _heads_tc(
        partials.reshape(NW, F), Wp, bp.reshape(1, E),
        Wv1, bv1.reshape(1, 64), Wv2, bv2.reshape(1, 32),
        Wv3, bv3.reshape(1, 16), Wv4.reshape(1, 16), bv4.reshape(1, 1))
    return policy.reshape(E), value.reshape(1)


# speedup vs baseline: 1.1020x; 1.0446x over previous
import functools

import jax
import jax.numpy as jnp
from jax import lax
from jax.experimental import pallas as pl
from jax.experimental.pallas import tpu as pltpu
from jax.experimental.pallas import tpu_sc as plsc

N = 1024
E = 16384
NC = 2
NS = 16
L = 16
NW = NC * NS
EPW = E // NW
G = EPW // L
F = 4 * N

TILE_E = 512
K_STEPS = E // TILE_E


def _edge_partials_sc(t, src, dst, a_flat, b_flat, w2s_flat):
    mesh = plsc.VectorSubcoreMesh(core_axis_name="c", subcore_axis_name="s",
                                  num_cores=NC, num_subcores=NS)

    @functools.partial(
        pl.kernel,
        out_type=jax.ShapeDtypeStruct((NW * F,), jnp.float32),
        mesh=mesh,
        compiler_params=pltpu.CompilerParams(needs_layout_passes=False),
        scratch_types=[
            pltpu.VMEM((N,), jnp.int32),
            pltpu.VMEM((EPW,), jnp.int32),
            pltpu.VMEM((EPW,), jnp.int32),
            pltpu.VMEM((N * L,), jnp.float32),
            pltpu.VMEM((N * L,), jnp.float32),
            pltpu.VMEM((48 * L,), jnp.float32),
            pltpu.VMEM((L * F,), jnp.float32),
            pltpu.VMEM((F,), jnp.float32),
            pltpu.SemaphoreType.DMA,
        ],
    )
    def sc_kernel(t_hbm, src_hbm, dst_hbm, a_hbm, b_hbm, w2_hbm, out_hbm,
                  t_v, src_v, dst_v, a_v, b_v, w2_v, acc_v, red_v, sem):
        wid = lax.axis_index("s") * NC + lax.axis_index("c")
        base = wid * EPW
        copies = [
            pltpu.async_copy(t_hbm, t_v, sem),
            pltpu.async_copy(src_hbm.at[pl.ds(base, EPW)], src_v, sem),
            pltpu.async_copy(dst_hbm.at[pl.ds(base, EPW)], dst_v, sem),
            pltpu.async_copy(a_hbm, a_v, sem),
            pltpu.async_copy(b_hbm, b_v, sem),
            pltpu.async_copy(w2_hbm, w2_v, sem),
        ]

        zero = jnp.zeros((L,), jnp.float32)

        def zbody(j, carry):
            for r in range(L):
                acc_v[pl.ds(j * (L * L) + r * L, L)] = zero
            return carry

        lax.fori_loop(0, (L * F) // (L * L), zbody, 0)
        for c in copies:
            c.wait()

        lanes = lax.iota(jnp.int32, L)
        lane_off = lanes * F
        lane_idx = lanes * G

        def bf16_round(x):
            bits = plsc.bitcast(x, jnp.int32)
            rounded = (bits + 0x7FFF + ((bits >> 16) & 1)) & ~0xFFFF
            return plsc.bitcast(rounded, jnp.float32)

        def tree_sum(ps):
            while len(ps) > 1:
                nxt = [ps[i] + ps[i + 1] for i in range(0, len(ps) - 1, 2)]
                if len(ps) % 2:
                    nxt.append(ps[-1])
                ps = nxt
            return ps[0]

        def gstep(g):
            ev = lane_idx + g
            srcv = plsc.load_gather(src_v, [ev])
            dstv = plsc.load_gather(dst_v, [ev])
            tdv = plsc.load_gather(t_v, [dstv]) * L
            tsv = plsc.load_gather(t_v, [srcv]) * L
            h = []
            for j in range(10):
                aj = plsc.load_gather(a_v, [tdv + j])
                bj = plsc.load_gather(b_v, [tsv + j])
                h.append(bf16_round(jnp.maximum(aj + bj, 0.0)))
            logit = []
            for c in range(4):
                w2c = [w2_v[pl.ds((j * 4 + c) * L, L)] for j in range(10)]
                prods = [h[j] * w2c[j] for j in range(10)]
                logit.append(tree_sum(prods) + w2_v[pl.ds((40 + c) * L, L)])
            m = jnp.maximum(jnp.maximum(logit[0], logit[1]),
                            jnp.maximum(logit[2], logit[3]))
            ex = [jnp.exp(x - m) for x in logit]
            inv = 1.0 / ((ex[0] + ex[1]) + (ex[2] + ex[3]))
            d4 = lane_off + dstv * 4
            for c in range(4):
                plsc.addupdate_scatter(acc_v, [d4 + c], ex[c] * inv)

        def gbody(i, carry):
            gstep(i * 2)
            gstep(i * 2 + 1)
            return carry

        lax.fori_loop(0, G // 2, gbody, 0)

        def rbody(j, carry):
            v = acc_v[pl.ds(j * L, L)]
            for r in range(1, L):
                v = v + acc_v[pl.ds(r * F + j * L, L)]
            red_v[pl.ds(j * L, L)] = v
            return carry

        lax.fori_loop(0, F // L, rbody, 0)
        pltpu.sync_copy(red_v, out_hbm.at[pl.ds(wid * F, F)])

    return sc_kernel(t, src, dst, a_flat, b_flat, w2s_flat)


def _heads_tc(partials, wp, bp2, wv1, bv1, wv2, bv2, wv3, bv3, wv4, bv4):
    def body(part_ref, wp_ref, bp_ref, wv1_ref, bv1_ref, wv2_ref, bv2_ref,
             wv3_ref, bv3_ref, wv4_ref, bv4_ref, pol_ref, val_ref, flat_s):
        k = pl.program_id(0)

        bf = jnp.bfloat16

        @pl.when(k == 0)
        def _():
            parts = part_ref[...]
            flat = parts[0:1, :]
            for w in range(1, NW):
                flat = flat + parts[w:w + 1, :]
            flat_s[...] = flat.astype(bf)
            v = jnp.maximum(jnp.dot(flat.astype(bf), wv1_ref[...].astype(bf),
                                    preferred_element_type=jnp.float32)
                            + bv1_ref[...], 0.0)
            v = jnp.maximum(jnp.dot(v.astype(bf), wv2_ref[...].astype(bf),
                                    preferred_element_type=jnp.float32)
                            + bv2_ref[...], 0.0)
            v = jnp.maximum(jnp.dot(v.astype(bf), wv3_ref[...].astype(bf),
                                    preferred_element_type=jnp.float32)
                            + bv3_ref[...], 0.0)
            prods = v * wv4_ref[...]
            ps = [prods[:, i:i + 1] for i in range(16)]
            while len(ps) > 1:
                ps = [ps[i] + ps[i + 1] for i in range(0, len(ps) - 1, 2)] \
                    + ([ps[-1]] if len(ps) % 2 else [])
            val_ref[...] = ps[0] + bv4_ref[...]

        logits = jnp.dot(flat_s[...], wp_ref[...].astype(bf),
                         preferred_element_type=jnp.float32) + bp_ref[...]
        pol_ref[:, pl.ds(k * TILE_E, TILE_E)] = logits

        @pl.when(k == K_STEPS - 1)
        def _():
            full = pol_ref[...]
            m = jnp.max(full)
            ex = jnp.exp(full - m)
            pol_ref[...] = ex / jnp.sum(ex)

    policy, value = pl.pallas_call(
        body,
        grid=(K_STEPS,),
        in_specs=[
            pl.BlockSpec((NW, F), lambda k: (0, 0)),
            pl.BlockSpec((F, TILE_E), lambda k: (0, k)),
            pl.BlockSpec((1, TILE_E), lambda k: (0, k)),
            pl.BlockSpec((F, 64), lambda k: (0, 0)),
            pl.BlockSpec((1, 64), lambda k: (0, 0)),
            pl.BlockSpec((64, 32), lambda k: (0, 0)),
            pl.BlockSpec((1, 32), lambda k: (0, 0)),
            pl.BlockSpec((32, 16), lambda k: (0, 0)),
            pl.BlockSpec((1, 16), lambda k: (0, 0)),
            pl.BlockSpec((1, 16), lambda k: (0, 0)),
            pl.BlockSpec((1, 1), lambda k: (0, 0)),
        ],
        out_specs=[
            pl.BlockSpec((1, E), lambda k: (0, 0)),
            pl.BlockSpec((1, 1), lambda k: (0, 0)),
        ],
        out_shape=[
            jax.ShapeDtypeStruct((1, E), jnp.float32),
            jax.ShapeDtypeStruct((1, 1), jnp.float32),
        ],
        scratch_shapes=[pltpu.VMEM((1, F), jnp.bfloat16)],
    )(partials, wp, bp2, wv1, bv1, wv2, bv2, wv3, bv3, wv4, bv4)
    return policy, value


def kernel(target_nodes, edge_index, W1, b1, W2, b2, Wp, bp,
           Wv1, bv1, Wv2, bv2, Wv3, bv3, Wv4, bv4):
    t = target_nodes.astype(jnp.int32)
    src = edge_index[0].astype(jnp.int32)
    dst = edge_index[1].astype(jnp.int32)

    bf = jnp.bfloat16
    w1a = W1[:N].astype(bf).astype(jnp.float32)
    w1b = W1[N:].astype(bf).astype(jnp.float32)
    a_flat = jnp.pad((w1a - w1b) + b1[None, :],
                     ((0, 0), (0, L - 10))).reshape(N * L)
    b_flat = jnp.pad(w1b, ((0, 0), (0, L - 10))).reshape(N * L)
    w2r = W2.astype(bf).astype(jnp.float32)
    w2s_flat = (jnp.concatenate(
        [w2r.reshape(40, 1), b2.reshape(4, 1), jnp.zeros((4, 1), jnp.float32)]
    ) * jnp.ones((1, L), jnp.float32)).reshape(48 * L)

    partials = _edge_partials_sc(t, src, dst, a_flat, b_flat, w2s_flat)

    policy, value = _heads_tc(
        partials.reshape(NW, F), Wp, bp.reshape(1, E),
        Wv1, bv1.reshape(1, 64), Wv2, bv2.reshape(1, 32),
        Wv3, bv3.reshape(1, 16), Wv4.reshape(1, 16), bv4.reshape(1, 1))
    return policy.reshape(E), value.reshape(1)
